# baseline (device time: 40186 ns/iter reference)
import jax
import jax.numpy as jnp
from jax import lax
from jax.experimental import pallas as pl
from jax.experimental.pallas import tpu as pltpu


def kernel(Q, K, V):
    b, q, h, d = Q.shape
    _, kv, _, _ = K.shape
    scale = d ** -0.5

    def body(q_ref, k_ref, v_ref, o_ref,
             u_send, u_recv, ml_send, ml_recv, send_sems, recv_sems):
        my_x = lax.axis_index("x")
        my_y = lax.axis_index("y")
        nbr = (1 - my_x, my_y)

        for bi in range(b):
            Qb = q_ref[bi, 0]
            Kb = k_ref[bi]
            Sb = jnp.sum(Qb[None, :, :] * Kb, axis=-1) * scale
            mb = jnp.max(Sb, axis=0, keepdims=True)
            Pb = jnp.exp(Sb - mb)
            lb = jnp.sum(Pb, axis=0, keepdims=True)
            Ub = jnp.sum(Pb[:, :, None] * v_ref[bi], axis=0)
            u_send[bi] = Ub
            ml_send[0, bi:bi + 1, :] = mb
            ml_send[1, bi:bi + 1, :] = lb

        barrier = pltpu.get_barrier_semaphore()
        pl.semaphore_signal(barrier, inc=1, device_id=nbr,
                            device_id_type=pl.DeviceIdType.MESH)
        pl.semaphore_wait(barrier, 1)

        rdma_u = pltpu.make_async_remote_copy(
            src_ref=u_send, dst_ref=u_recv,
            send_sem=send_sems.at[0], recv_sem=recv_sems.at[0],
            device_id=nbr, device_id_type=pl.DeviceIdType.MESH,
        )
        rdma_ml = pltpu.make_async_remote_copy(
            src_ref=ml_send, dst_ref=ml_recv,
            send_sem=send_sems.at[1], recv_sem=recv_sems.at[1],
            device_id=nbr, device_id_type=pl.DeviceIdType.MESH,
        )
        rdma_u.start()
        rdma_ml.start()
        rdma_u.wait()
        rdma_ml.wait()

        m_loc = ml_send[0]
        l_loc = ml_send[1]
        m_rem = ml_recv[0]
        l_rem = ml_recv[1]
        m_new = jnp.maximum(m_loc, m_rem)
        a = jnp.exp(m_loc - m_new)
        c = jnp.exp(m_rem - m_new)
        l_new = l_loc * a + l_rem * c
        U = u_send[:] * a[:, :, None] + u_recv[:] * c[:, :, None]
        o_ref[:, 0, :, :] = U / l_new[:, :, None]

    return pl.pallas_call(
        body,
        out_shape=jax.ShapeDtypeStruct((b, q, h, d), jnp.float32),
        in_specs=[
            pl.BlockSpec(memory_space=pltpu.VMEM),
            pl.BlockSpec(memory_space=pltpu.VMEM),
            pl.BlockSpec(memory_space=pltpu.VMEM),
        ],
        out_specs=pl.BlockSpec(memory_space=pltpu.VMEM),
        scratch_shapes=[
            pltpu.VMEM((b, h, d), jnp.float32),
            pltpu.VMEM((b, h, d), jnp.float32),
            pltpu.VMEM((2, b, h), jnp.float32),
            pltpu.VMEM((2, b, h), jnp.float32),
            pltpu.SemaphoreType.DMA((2,)),
            pltpu.SemaphoreType.DMA((2,)),
        ],
        compiler_params=pltpu.CompilerParams(collective_id=0),
    )(Q, K, V)


# device time: 21744 ns/iter; 1.8481x vs baseline; 1.8481x over previous
import jax
import jax.numpy as jnp
from jax import lax
from jax.experimental import pallas as pl
from jax.experimental.pallas import tpu as pltpu


def kernel(Q, K, V):
    b, q, h, d = Q.shape
    _, kv, _, _ = K.shape
    hd = h * d
    scale = d ** -0.5

    Q2 = Q.reshape(b, hd, 1)
    K2 = K.reshape(b, kv, hd)
    V2 = V.reshape(b, kv, hd)

    def body(q_ref, k_ref, v_ref, o_ref,
             u_send, u_recv, ml_send, ml_recv, send_sems, recv_sems):
        my_x = lax.axis_index("x")
        my_y = lax.axis_index("y")
        nbr = (1 - my_x, my_y)

        rows = lax.broadcasted_iota(jnp.int32, (hd, h), 0) // d
        cols = lax.broadcasted_iota(jnp.int32, (hd, h), 1)
        sel = (rows == cols)
        onehot_t = jnp.where(sel, 1.0, 0.0).astype(jnp.float32).T

        for bi in range(b):
            Qblk = jnp.where(sel, q_ref[bi], 0.0)
            Sb = jnp.dot(k_ref[bi], Qblk,
                         preferred_element_type=jnp.float32) * scale
            mb = jnp.max(Sb, axis=0, keepdims=True)
            Pb = jnp.exp(Sb - mb)
            lb = jnp.sum(Pb, axis=0, keepdims=True)
            Pexp = jnp.dot(Pb, onehot_t,
                           preferred_element_type=jnp.float32)
            Ub = jnp.sum(Pexp * v_ref[bi], axis=0, keepdims=True)
            u_send[bi:bi + 1, :] = Ub
            ml_send[0, bi:bi + 1, :] = mb
            ml_send[1, bi:bi + 1, :] = lb

        barrier = pltpu.get_barrier_semaphore()
        pl.semaphore_signal(barrier, inc=1, device_id=nbr,
                            device_id_type=pl.DeviceIdType.MESH)
        pl.semaphore_wait(barrier, 1)

        rdma_u = pltpu.make_async_remote_copy(
            src_ref=u_send, dst_ref=u_recv,
            send_sem=send_sems.at[0], recv_sem=recv_sems.at[0],
            device_id=nbr, device_id_type=pl.DeviceIdType.MESH,
        )
        rdma_ml = pltpu.make_async_remote_copy(
            src_ref=ml_send, dst_ref=ml_recv,
            send_sem=send_sems.at[1], recv_sem=recv_sems.at[1],
            device_id=nbr, device_id_type=pl.DeviceIdType.MESH,
        )
        rdma_u.start()
        rdma_ml.start()
        rdma_u.wait()
        rdma_ml.wait()

        m_loc = ml_send[0]
        l_loc = ml_send[1]
        m_rem = ml_recv[0]
        l_rem = ml_recv[1]
        m_new = jnp.maximum(m_loc, m_rem)
        a = jnp.exp(m_loc - m_new)
        c = jnp.exp(m_rem - m_new)
        l_new = l_loc * a + l_rem * c
        a_exp = jnp.dot(a, onehot_t, preferred_element_type=jnp.float32)
        c_exp = jnp.dot(c, onehot_t, preferred_element_type=jnp.float32)
        l_exp = jnp.dot(l_new, onehot_t, preferred_element_type=jnp.float32)
        o_ref[:, :] = (u_send[:, :] * a_exp + u_recv[:, :] * c_exp) / l_exp

    out = pl.pallas_call(
        body,
        out_shape=jax.ShapeDtypeStruct((b, hd), jnp.float32),
        in_specs=[
            pl.BlockSpec(memory_space=pltpu.VMEM),
            pl.BlockSpec(memory_space=pltpu.VMEM),
            pl.BlockSpec(memory_space=pltpu.VMEM),
        ],
        out_specs=pl.BlockSpec(memory_space=pltpu.VMEM),
        scratch_shapes=[
            pltpu.VMEM((b, hd), jnp.float32),
            pltpu.VMEM((b, hd), jnp.float32),
            pltpu.VMEM((2, b, h), jnp.float32),
            pltpu.VMEM((2, b, h), jnp.float32),
            pltpu.SemaphoreType.DMA((2,)),
            pltpu.SemaphoreType.DMA((2,)),
        ],
        compiler_params=pltpu.CompilerParams(collective_id=0),
    )(Q2, K2, V2)
    return out.reshape(b, q, h, d)
